# baseline (speedup 1.0000x reference)
import functools

import jax
import jax.numpy as jnp
from jax import lax
from jax.experimental import pallas as pl
from jax.experimental.pallas import tpu as pltpu
from jax.experimental.pallas import tpu_sc as plsc

NO_CAT = 1000000
EMBED_DIM = 32
BATCH = 16384

_PACK = 128 // EMBED_DIM
_ROWS4 = NO_CAT // _PACK

_info = plsc.get_sparse_core_info()
_NC = _info.num_cores
_NS = _info.num_subcores
_NW = _NC * _NS
_B_W = BATCH // _NW
_CHUNK = 128
_N_CHUNKS = _B_W // _CHUNK
_L = 16
_BLK_PER_CHUNK = _CHUNK // _L

_mesh = plsc.VectorSubcoreMesh(core_axis_name="c", subcore_axis_name="s")


@functools.partial(
    pl.kernel,
    mesh=_mesh,
    out_type=jax.ShapeDtypeStruct((BATCH, EMBED_DIM), jnp.float32),
    compiler_params=pltpu.CompilerParams(needs_layout_passes=False),
    scratch_types=[
        pltpu.VMEM((_B_W,), jnp.int32),
        pltpu.VMEM((_B_W,), jnp.int32),
        pltpu.VMEM((2, _CHUNK, 4 * EMBED_DIM), jnp.float32),
        pltpu.VMEM((_B_W, EMBED_DIM), jnp.float32),
        pltpu.SemaphoreType.DMA,
        pltpu.SemaphoreType.DMA,
    ],
)
def _embed_gather(x_hbm, table4_hbm, out_hbm, idx_v, q_v, rows4_v, out_v,
                  sem0, sem1):
    wid = lax.axis_index("s") * _NC + lax.axis_index("c")
    base = wid * _B_W
    pltpu.sync_copy(x_hbm.at[pl.ds(base, _B_W)], idx_v)

    def compute_q(i, _):
        v = idx_v[pl.ds(i * _L, _L)]
        q_v[pl.ds(i * _L, _L)] = lax.shift_right_logical(v, 2)
        return 0

    lax.fori_loop(0, _B_W // _L, compute_q, 0)

    sems = [sem0, sem1]

    def fire(j):
        return pltpu.async_copy(
            table4_hbm.at[q_v.at[pl.ds(j * _CHUNK, _CHUNK)]],
            rows4_v.at[j % 2],
            sems[j % 2],
        )

    iota = lax.iota(jnp.int32, _L)

    pending = fire(0)
    for j in range(_N_CHUNKS):
        pending.wait()
        if j + 1 < _N_CHUNKS:
            pending = fire(j + 1)
        buf = rows4_v.at[j % 2]

        def select_block(k, _, j=j, buf=buf):
            lrow = iota + k * _L
            grow = lrow + j * _CHUNK
            r = idx_v[pl.ds(j * _CHUNK + k * _L, _L)] & (_PACK - 1)
            col0 = r * EMBED_DIM
            for d in range(EMBED_DIM):
                vals = plsc.load_gather(buf, [lrow, col0 + d])
                plsc.store_scatter(
                    out_v, [grow, jnp.full((_L,), d, jnp.int32)], vals)
            return 0

        lax.fori_loop(0, _BLK_PER_CHUNK, select_block, 0)

    pltpu.sync_copy(out_v, out_hbm.at[pl.ds(base, _B_W)])


def kernel(x, table):
    table4 = table.reshape(_ROWS4, 4 * EMBED_DIM)
    return _embed_gather(x.astype(jnp.int32), table4)


# speedup vs baseline: 3.9260x; 3.9260x over previous
import functools

import jax
import jax.numpy as jnp
from jax import lax
from jax.experimental import pallas as pl
from jax.experimental.pallas import tpu as pltpu
from jax.experimental.pallas import tpu_sc as plsc

NO_CAT = 1000000
EMBED_DIM = 32
BATCH = 16384

_L = 16
_TAIL_COL0 = (NO_CAT // 128 - 2) * 128
_TAIL_W = NO_CAT - _TAIL_COL0

_info = plsc.get_sparse_core_info()
_NC = _info.num_cores
_NS = _info.num_subcores
_NW = _NC * _NS
_B_W = BATCH // _NW
_NCHUNK = _B_W // _L

_mesh = plsc.VectorSubcoreMesh(core_axis_name="c", subcore_axis_name="s")


@functools.partial(
    pl.kernel,
    mesh=_mesh,
    out_type=jax.ShapeDtypeStruct((EMBED_DIM, BATCH), jnp.float32),
    compiler_params=pltpu.CompilerParams(needs_layout_passes=False),
    scratch_types=[
        pltpu.VMEM((_B_W,), jnp.int32),
        pltpu.VMEM((_L, EMBED_DIM, 128), jnp.float32),
        pltpu.VMEM((EMBED_DIM, _B_W), jnp.float32),
        pltpu.VMEM((EMBED_DIM, _TAIL_W), jnp.float32),
        pltpu.SemaphoreType.DMA,
        pltpu.SemaphoreType.DMA,
    ],
)
def _embed_gather(x_hbm, tabt_hbm, outt_hbm, idx_v, tiles_v, out_v, tail_v,
                  sem, tail_sem):
    wid = lax.axis_index("s") * _NC + lax.axis_index("c")
    b0 = wid * _B_W

    tail_cp = pltpu.async_copy(
        tabt_hbm.at[:, pl.ds(_TAIL_COL0, _TAIL_W)], tail_v, tail_sem)
    pltpu.sync_copy(x_hbm.at[pl.ds(b0, _B_W)], idx_v)
    tail_cp.wait()

    iota = lax.iota(jnp.int32, _L)

    def chunk_body(k, _):
        ivec = idx_v[pl.ds(k * _L, _L)]
        tmask = ivec >= _TAIL_COL0
        isafe = jnp.where(tmask, 0, ivec)
        copies = []
        for l in range(_L):
            g = lax.reduce_max(jnp.where(iota == l, isafe, 0), (0,))
            col = pl.multiple_of(lax.shift_left(
                lax.shift_right_logical(g, 7), 7), 128)
            copies.append(pltpu.async_copy(
                tabt_hbm.at[:, pl.ds(col, 128)], tiles_v.at[l], sem))
        for cp in copies:
            cp.wait()
        lane = isafe & 127
        tcol = jnp.maximum(ivec - _TAIL_COL0, 0)
        for d in range(EMBED_DIM):
            drow = jnp.full((_L,), d, jnp.int32)
            vals = plsc.load_gather(tiles_v, [iota, drow, lane])
            tvals = plsc.load_gather(tail_v, [drow, tcol], mask=tmask)
            out_v[d, pl.ds(k * _L, _L)] = jnp.where(tmask, tvals, vals)
        return 0

    lax.fori_loop(0, _NCHUNK, chunk_body, 0)

    pltpu.sync_copy(out_v, outt_hbm.at[:, pl.ds(b0, _B_W)])


def kernel(x, table):
    outt = _embed_gather(x.astype(jnp.int32), table.T)
    return outt.T
